# baseline (speedup 1.0000x reference)
import functools

import jax
import jax.numpy as jnp
from jax import lax
from jax.experimental import pallas as pl
from jax.experimental.pallas import tpu as pltpu
from jax.experimental.pallas import tpu_sc as plsc

N = 10000
E = 160000
F = 256
H = 128
NPAD = 10240
EPAD = 163840
CK = 64
CHUNKS = EPAD // CK
CPT = CHUNKS // 16
HSTG = 20
NBATCH = CPT // HSTG
NBANK = 3
EPW = EPAD // 32
RPT = NPAD // 16
SINK = N

_mesh = plsc.VectorSubcoreMesh(core_axis_name="c", subcore_axis_name="s")
_f32 = jnp.float32
_sc_params = pltpu.CompilerParams(
    needs_layout_passes=False, use_tc_tiling_on_sc=False)


@functools.partial(
    pl.kernel,
    out_type=(
        jax.ShapeDtypeStruct((32 * NPAD,), _f32),
        jax.ShapeDtypeStruct((32 * NPAD,), _f32),
    ),
    mesh=_mesh,
    scratch_types=[
        pltpu.VMEM((EPW,), jnp.int32),
        pltpu.VMEM((EPW,), jnp.int32),
        pltpu.VMEM((NPAD,), _f32),
        pltpu.VMEM((NPAD,), _f32),
    ],
    compiler_params=_sc_params,
)
def _deg_kernel(src_hbm, dst_hbm, outs_hbm, outd_hbm, src_v, dst_v, degs, degd):
    cid = lax.axis_index("c")
    sid = lax.axis_index("s")
    w = cid * 16 + sid

    def zero(i, carry):
        z = jnp.zeros((16,), _f32)
        degs[pl.ds(i * 16, 16)] = z
        degd[pl.ds(i * 16, 16)] = z
        return carry

    lax.fori_loop(0, NPAD // 16, zero, 0)

    pltpu.sync_copy(src_hbm.at[pl.ds(w * EPW, EPW)], src_v)
    pltpu.sync_copy(dst_hbm.at[pl.ds(w * EPW, EPW)], dst_v)

    ones = jnp.full((16,), 1.0, _f32)

    def body(v, carry):
        sv = src_v[pl.ds(v * 16, 16)]
        dv = dst_v[pl.ds(v * 16, 16)]
        plsc.addupdate_scatter(degs, [sv], ones)
        plsc.addupdate_scatter(degd, [dv], ones)
        return carry

    lax.fori_loop(0, EPW // 16, body, 0)

    pltpu.sync_copy(degs, outs_hbm.at[pl.ds(w * NPAD, NPAD)])
    pltpu.sync_copy(degd, outd_hbm.at[pl.ds(w * NPAD, NPAD)])


def _normx_body(x_ref, dps_ref, dpd_ref, x0_ref, x1_ref, ns_ref, nd_ref):
    degs = jnp.sum(jnp.transpose(dps_ref[...]), axis=1, keepdims=True)
    degd = jnp.sum(jnp.transpose(dpd_ref[...]), axis=1, keepdims=True)
    ns = lax.rsqrt(jnp.maximum(degs, 1.0))
    nd = lax.rsqrt(jnp.maximum(degd, 1.0))
    xn = x_ref[...] * ns
    x0_ref[...] = xn[:, 0 * H:1 * H]
    x1_ref[...] = xn[:, 1 * H:2 * H]
    ns_ref[...] = ns
    nd_ref[...] = nd


def _normx(x_pad, dps, dpd):
    return pl.pallas_call(
        _normx_body,
        out_shape=(
            jax.ShapeDtypeStruct((NPAD, H), _f32),
            jax.ShapeDtypeStruct((NPAD, H), _f32),
            jax.ShapeDtypeStruct((NPAD, 1), _f32),
            jax.ShapeDtypeStruct((NPAD, 1), _f32),
        ),
    )(x_pad, dps, dpd)


@functools.partial(
    pl.kernel,
    out_type=(
        jax.ShapeDtypeStruct((NPAD, H), _f32),
        jax.ShapeDtypeStruct((NPAD, H), _f32),
        jax.ShapeDtypeStruct((32 * NPAD,), _f32),
    ),
    mesh=_mesh,
    scratch_types=[
        pltpu.VMEM((HSTG, CK), jnp.int32),
        pltpu.VMEM((HSTG, CK), jnp.int32),
    ] + [pltpu.VMEM((CK, H), _f32)] * NBANK + [
        pltpu.VMEM((NPAD,), _f32),
        pltpu.VMEM((NPAD,), _f32),
        pltpu.VMEM_SHARED((NPAD, H), _f32),
    ] + [pltpu.SemaphoreType.DMA] * (2 * NBANK),
    compiler_params=_sc_params,
)
def _agg_kernel(src2_hbm, dst2_hbm, x0_hbm, x1_hbm, nd_hbm,
                agg0_hbm, agg1_hbm, cpart_hbm,
                src_v, dst_v, r0, r1, r2,
                nd_v, c_v, acc_sh,
                sem_g0, sem_g1, sem_g2,
                sem_s0, sem_s1, sem_s2):
    rows = (r0, r1, r2)
    sem_g = (sem_g0, sem_g1, sem_g2)
    sem_s = (sem_s0, sem_s1, sem_s2)
    cid = lax.axis_index("c")
    sid = lax.axis_index("s")

    def zc(i, carry):
        c_v[pl.ds(i * 16, 16)] = jnp.zeros((16,), _f32)
        return carry

    lax.fori_loop(0, NPAD // 16, zc, 0)
    pltpu.sync_copy(nd_hbm, nd_v)

    def c_work(j):
        for k in range(CK // 16):
            dvec = dst_v[j, pl.ds(k * 16, 16)]
            svec = src_v[j, pl.ds(k * 16, 16)]
            vals = plsc.load_gather(nd_v, [dvec])
            plsc.addupdate_scatter(c_v, [svec], vals)

    def zrow(r, carry):
        for k in range(H // 16):
            r0[r, pl.ds(k * 16, 16)] = jnp.zeros((16,), _f32)
        return carry

    lax.fori_loop(0, CK, zrow, 0)
    for q in range(RPT // CK):
        pltpu.sync_copy(r0, acc_sh.at[pl.ds(sid * RPT + q * CK, CK)])
    plsc.subcore_barrier()

    for c in range(2):

        @pl.when(cid == c)
        def _(cpar=c, x_hbm=(x0_hbm, x1_hbm)[c]):
            for h in range(NBATCH):
                pltpu.sync_copy(
                    src2_hbm.at[pl.ds(sid * CPT + h * HSTG, HSTG)], src_v)
                pltpu.sync_copy(
                    dst2_hbm.at[pl.ds(sid * CPT + h * HSTG, HSTG)], dst_v)

                for i in range(NBANK):
                    pltpu.async_copy(
                        x_hbm.at[src_v.at[i]], rows[i], sem_g[i])

                for j in range(HSTG):
                    bank = j % NBANK
                    pltpu.make_async_copy(
                        x_hbm.at[src_v.at[j]],
                        rows[bank], sem_g[bank]).wait()
                    pltpu.async_copy(
                        rows[bank],
                        acc_sh.at[dst_v.at[j]],
                        sem_s[bank], add=True)
                    if cpar == j % 2:
                        c_work(j)
                    pltpu.make_async_copy(
                        rows[bank],
                        acc_sh.at[dst_v.at[j]],
                        sem_s[bank]).wait()
                    if j + NBANK < HSTG:
                        pltpu.async_copy(
                            x_hbm.at[src_v.at[j + NBANK]],
                            rows[bank], sem_g[bank])

    plsc.subcore_barrier()

    for c in range(2):

        @pl.when(cid == c)
        def _(agg_hbm=(agg0_hbm, agg1_hbm)[c]):
            pltpu.sync_copy(acc_sh.at[pl.ds(sid * RPT, RPT)],
                            agg_hbm.at[pl.ds(sid * RPT, RPT)])

    w = cid * 16 + sid
    pltpu.sync_copy(c_v, cpart_hbm.at[pl.ds(w * NPAD, NPAD)])


def _head_body(agg0_ref, agg1_ref, cpart_ref, ns_ref,
               nd_ref, w1_ref, b1_ref, w2_ref, b2_ref, out_ref):
    agg = jnp.concatenate([agg0_ref[...], agg1_ref[...]], axis=1)
    h = agg * nd_ref[...]
    h1 = jnp.maximum(
        jnp.dot(h, w1_ref[...], preferred_element_type=_f32)
        + b1_ref[...][None, :], 0.0)
    c = jnp.sum(jnp.transpose(cpart_ref[...]), axis=1, keepdims=True)
    w = ns_ref[...] * c
    rows = lax.broadcasted_iota(jnp.int32, (NPAD, 1), 0)
    w = jnp.where(rows < N, w, 0.0)
    v = jnp.sum(h1 * w, axis=0, keepdims=True) * (1.0 / N)
    out_ref[...] = (jnp.dot(v, w2_ref[...], preferred_element_type=_f32)
                    + b2_ref[...][None, :])


def _head(aggs, cpart, ns, nd, w1, b1, w2, b2):
    return pl.pallas_call(
        _head_body,
        out_shape=jax.ShapeDtypeStruct((1, F), _f32),
    )(*aggs, cpart, ns, nd, w1, b1, w2, b2)


def kernel(in_feat, edge_index, W1, b1, W2, b2):
    src = edge_index[0].astype(jnp.int32)
    dst = edge_index[1].astype(jnp.int32)
    sinks = jnp.full((EPAD - E,), SINK, jnp.int32)
    src_p = jnp.concatenate([src, sinks])
    dst_p = jnp.concatenate([dst, sinks])
    x_pad = jnp.pad(in_feat.astype(_f32), ((0, NPAD - N), (0, 0)))

    dps, dpd = _deg_kernel(src_p, dst_p)
    x0, x1, ns, nd = _normx(
        x_pad, dps.reshape(32, NPAD), dpd.reshape(32, NPAD))
    agg0, agg1, cpart = _agg_kernel(
        src_p.reshape(CHUNKS, CK), dst_p.reshape(CHUNKS, CK),
        x0, x1, nd.reshape(NPAD))
    return _head((agg0, agg1), cpart.reshape(32, NPAD),
                 ns, nd, W1, b1, W2, b2)


# speedup vs baseline: 1.0391x; 1.0391x over previous
import functools

import jax
import jax.numpy as jnp
from jax import lax
from jax.experimental import pallas as pl
from jax.experimental.pallas import tpu as pltpu
from jax.experimental.pallas import tpu_sc as plsc

N = 10000
E = 160000
F = 256
H = 128
NPAD = 10240
EPAD = 163840
CK = 32
CHUNKS = EPAD // CK
CPT = CHUNKS // 16
HSTG = CPT // 2
EPW = EPAD // 32
RPT = NPAD // 16
SINK = N

_mesh = plsc.VectorSubcoreMesh(core_axis_name="c", subcore_axis_name="s")
_f32 = jnp.float32
_sc_params = pltpu.CompilerParams(
    needs_layout_passes=False, use_tc_tiling_on_sc=False)


@functools.partial(
    pl.kernel,
    out_type=(
        jax.ShapeDtypeStruct((32 * NPAD,), _f32),
        jax.ShapeDtypeStruct((32 * NPAD,), _f32),
    ),
    mesh=_mesh,
    scratch_types=[
        pltpu.VMEM((EPW,), jnp.int32),
        pltpu.VMEM((EPW,), jnp.int32),
        pltpu.VMEM((NPAD,), _f32),
        pltpu.VMEM((NPAD,), _f32),
    ],
    compiler_params=_sc_params,
)
def _deg_kernel(src_hbm, dst_hbm, outs_hbm, outd_hbm, src_v, dst_v, degs, degd):
    cid = lax.axis_index("c")
    sid = lax.axis_index("s")
    w = cid * 16 + sid

    def zero(i, carry):
        z = jnp.zeros((16,), _f32)
        degs[pl.ds(i * 16, 16)] = z
        degd[pl.ds(i * 16, 16)] = z
        return carry

    lax.fori_loop(0, NPAD // 16, zero, 0)

    pltpu.sync_copy(src_hbm.at[pl.ds(w * EPW, EPW)], src_v)
    pltpu.sync_copy(dst_hbm.at[pl.ds(w * EPW, EPW)], dst_v)

    ones = jnp.full((16,), 1.0, _f32)

    def body(v, carry):
        sv = src_v[pl.ds(v * 16, 16)]
        dv = dst_v[pl.ds(v * 16, 16)]
        plsc.addupdate_scatter(degs, [sv], ones)
        plsc.addupdate_scatter(degd, [dv], ones)
        return carry

    lax.fori_loop(0, EPW // 16, body, 0)

    pltpu.sync_copy(degs, outs_hbm.at[pl.ds(w * NPAD, NPAD)])
    pltpu.sync_copy(degd, outd_hbm.at[pl.ds(w * NPAD, NPAD)])


def _normx_body(x_ref, dps_ref, dpd_ref, x0_ref, x1_ref, ns_ref, nd_ref):
    degs = jnp.sum(jnp.transpose(dps_ref[...]), axis=1, keepdims=True)
    degd = jnp.sum(jnp.transpose(dpd_ref[...]), axis=1, keepdims=True)
    ns = lax.rsqrt(jnp.maximum(degs, 1.0))
    nd = lax.rsqrt(jnp.maximum(degd, 1.0))
    xn = x_ref[...] * ns
    x0_ref[...] = xn[:, 0 * H:1 * H]
    x1_ref[...] = xn[:, 1 * H:2 * H]
    ns_ref[...] = ns
    nd_ref[...] = nd


def _normx(x_pad, dps, dpd):
    return pl.pallas_call(
        _normx_body,
        out_shape=(
            jax.ShapeDtypeStruct((NPAD, H), _f32),
            jax.ShapeDtypeStruct((NPAD, H), _f32),
            jax.ShapeDtypeStruct((NPAD, 1), _f32),
            jax.ShapeDtypeStruct((NPAD, 1), _f32),
        ),
    )(x_pad, dps, dpd)


@functools.partial(
    pl.kernel,
    out_type=(
        jax.ShapeDtypeStruct((NPAD, H), _f32),
        jax.ShapeDtypeStruct((NPAD, H), _f32),
        jax.ShapeDtypeStruct((32 * NPAD,), _f32),
    ),
    mesh=_mesh,
    scratch_types=[
        pltpu.VMEM((HSTG, CK), jnp.int32),
        pltpu.VMEM((HSTG, CK), jnp.int32),
    ] + [pltpu.VMEM((CK, H), _f32)] * 4 + [
        pltpu.VMEM((NPAD,), _f32),
        pltpu.VMEM((NPAD,), _f32),
        pltpu.VMEM_SHARED((NPAD, H), _f32),
    ] + [pltpu.SemaphoreType.DMA] * 8,
    compiler_params=_sc_params,
)
def _agg_kernel(src2_hbm, dst2_hbm, x0_hbm, x1_hbm, nd_hbm,
                agg0_hbm, agg1_hbm, cpart_hbm,
                src_v, dst_v, r0, r1, r2, r3,
                nd_v, c_v, acc_sh,
                sem_g0, sem_g1, sem_g2, sem_g3,
                sem_s0, sem_s1, sem_s2, sem_s3):
    rows = (r0, r1, r2, r3)
    sem_g = (sem_g0, sem_g1, sem_g2, sem_g3)
    sem_s = (sem_s0, sem_s1, sem_s2, sem_s3)
    cid = lax.axis_index("c")
    sid = lax.axis_index("s")

    def zc(i, carry):
        c_v[pl.ds(i * 16, 16)] = jnp.zeros((16,), _f32)
        return carry

    lax.fori_loop(0, NPAD // 16, zc, 0)
    pltpu.sync_copy(nd_hbm, nd_v)

    def c_work(j):
        for k in range(CK // 16):
            dvec = dst_v[j, pl.ds(k * 16, 16)]
            svec = src_v[j, pl.ds(k * 16, 16)]
            vals = plsc.load_gather(nd_v, [dvec])
            plsc.addupdate_scatter(c_v, [svec], vals)

    def zrow(r, carry):
        for k in range(H // 16):
            r0[r, pl.ds(k * 16, 16)] = jnp.zeros((16,), _f32)
        return carry

    lax.fori_loop(0, CK, zrow, 0)
    for q in range(RPT // CK):
        pltpu.sync_copy(r0, acc_sh.at[pl.ds(sid * RPT + q * CK, CK)])
    plsc.subcore_barrier()

    for c in range(2):

        @pl.when(cid == c)
        def _(cpar=c, x_hbm=(x0_hbm, x1_hbm)[c]):
            for h in range(2):
                pltpu.sync_copy(
                    src2_hbm.at[pl.ds(sid * CPT + h * HSTG, HSTG)], src_v)
                pltpu.sync_copy(
                    dst2_hbm.at[pl.ds(sid * CPT + h * HSTG, HSTG)], dst_v)

                for i in range(4):
                    pltpu.async_copy(
                        x_hbm.at[src_v.at[i]], rows[i], sem_g[i])

                def quad(t, carry):
                    base = 4 * t
                    for bank in range(4):
                        pltpu.make_async_copy(
                            x_hbm.at[src_v.at[base + bank]],
                            rows[bank], sem_g[bank]).wait()
                        pltpu.async_copy(
                            rows[bank],
                            acc_sh.at[dst_v.at[base + bank]],
                            sem_s[bank], add=True)
                        if cpar == bank % 2:
                            c_work(base + bank)
                        pltpu.make_async_copy(
                            rows[bank],
                            acc_sh.at[dst_v.at[base + bank]],
                            sem_s[bank]).wait()

                        @pl.when(t < HSTG // 4 - 1)
                        def _(bank=bank):
                            pltpu.async_copy(
                                x_hbm.at[src_v.at[base + bank + 4]],
                                rows[bank], sem_g[bank])
                    return carry

                lax.fori_loop(0, HSTG // 4, quad, 0)

    plsc.subcore_barrier()

    for c in range(2):

        @pl.when(cid == c)
        def _(agg_hbm=(agg0_hbm, agg1_hbm)[c]):
            pltpu.sync_copy(acc_sh.at[pl.ds(sid * RPT, RPT)],
                            agg_hbm.at[pl.ds(sid * RPT, RPT)])

    w = cid * 16 + sid
    pltpu.sync_copy(c_v, cpart_hbm.at[pl.ds(w * NPAD, NPAD)])


def _head_body(agg0_ref, agg1_ref, cpart_ref, ns_ref,
               nd_ref, w1_ref, b1_ref, w2_ref, b2_ref, out_ref):
    agg = jnp.concatenate([agg0_ref[...], agg1_ref[...]], axis=1)
    h = agg * nd_ref[...]
    h1 = jnp.maximum(
        jnp.dot(h, w1_ref[...], preferred_element_type=_f32)
        + b1_ref[...][None, :], 0.0)
    c = jnp.sum(jnp.transpose(cpart_ref[...]), axis=1, keepdims=True)
    w = ns_ref[...] * c
    rows = lax.broadcasted_iota(jnp.int32, (NPAD, 1), 0)
    w = jnp.where(rows < N, w, 0.0)
    v = jnp.sum(h1 * w, axis=0, keepdims=True) * (1.0 / N)
    out_ref[...] = (jnp.dot(v, w2_ref[...], preferred_element_type=_f32)
                    + b2_ref[...][None, :])


def _head(aggs, cpart, ns, nd, w1, b1, w2, b2):
    return pl.pallas_call(
        _head_body,
        out_shape=jax.ShapeDtypeStruct((1, F), _f32),
    )(*aggs, cpart, ns, nd, w1, b1, w2, b2)


def kernel(in_feat, edge_index, W1, b1, W2, b2):
    src = edge_index[0].astype(jnp.int32)
    dst = edge_index[1].astype(jnp.int32)
    sinks = jnp.full((EPAD - E,), SINK, jnp.int32)
    src_p = jnp.concatenate([src, sinks])
    dst_p = jnp.concatenate([dst, sinks])
    x_pad = jnp.pad(in_feat.astype(_f32), ((0, NPAD - N), (0, 0)))

    dps, dpd = _deg_kernel(src_p, dst_p)
    x0, x1, ns, nd = _normx(
        x_pad, dps.reshape(32, NPAD), dpd.reshape(32, NPAD))
    agg0, agg1, cpart = _agg_kernel(
        src_p.reshape(CHUNKS, CK), dst_p.reshape(CHUNKS, CK),
        x0, x1, nd.reshape(NPAD))
    return _head((agg0, agg1), cpart.reshape(32, NPAD),
                 ns, nd, W1, b1, W2, b2)
